# baseline (speedup 1.0000x reference)
import functools

import jax
import jax.numpy as jnp
from jax import lax
from jax.experimental import pallas as pl
from jax.experimental.pallas import tpu as pltpu
from jax.experimental.pallas import tpu_sc as plsc

_EMBED = 64
_BATCH = 16384
_ROWS = 1000000
_CHUNK = 16
_NCHUNK = None


def _build():
    info = plsc.get_sparse_core_info()
    nc, ns, nl = info.num_cores, info.num_subcores, info.num_lanes
    nw = nc * ns
    b_per_w = _BATCH // nw
    n_chunks = b_per_w // _CHUNK
    idx_rows_per_w = b_per_w // 128
    mesh = plsc.VectorSubcoreMesh(core_axis_name="c", subcore_axis_name="s")

    @functools.partial(
        pl.kernel,
        mesh=mesh,
        out_type=jax.ShapeDtypeStruct((_BATCH, _EMBED), jnp.float32),
        scratch_types=[
            pltpu.VMEM((idx_rows_per_w, 128), jnp.int32),
            pltpu.VMEM((idx_rows_per_w, 128), jnp.int32),
            pltpu.VMEM((idx_rows_per_w, 128), jnp.int32),
            pltpu.VMEM((idx_rows_per_w, 128), jnp.int32),
            pltpu.VMEM((_CHUNK * 8, _EMBED), jnp.float32),
            pltpu.VMEM((_CHUNK * 8, _EMBED), jnp.float32),
            pltpu.VMEM((_CHUNK * 8, _EMBED), jnp.float32),
            pltpu.VMEM((_CHUNK * 8, _EMBED), jnp.float32),
            pltpu.VMEM((_CHUNK, _EMBED), jnp.float32),
            pltpu.VMEM((_CHUNK, _EMBED), jnp.float32),
            pltpu.SemaphoreType.DMA,
            pltpu.SemaphoreType.DMA,
            pltpu.SemaphoreType.DMA,
            pltpu.SemaphoreType.DMA,
        ],
    )
    def gmf(ublk_hbm, usub_hbm, iblk_hbm, isub_hbm, utab_hbm, itab_hbm, out_hbm,
            ublk_v, usub_v, iblk_v, isub_v,
            ub0, ub1, ib0, ib1, prod0, prod1,
            gsem0, gsem1, osem0, osem1):
        wid = lax.axis_index("s") * nc + lax.axis_index("c")
        base = wid * b_per_w
        irow0 = wid * idx_rows_per_w
        pltpu.sync_copy(ublk_hbm.at[pl.ds(irow0, idx_rows_per_w)], ublk_v)
        pltpu.sync_copy(usub_hbm.at[pl.ds(irow0, idx_rows_per_w)], usub_v)
        pltpu.sync_copy(iblk_hbm.at[pl.ds(irow0, idx_rows_per_w)], iblk_v)
        pltpu.sync_copy(isub_hbm.at[pl.ds(irow0, idx_rows_per_w)], isub_v)

        ub = (ub0, ub1)
        ib = (ib0, ib1)
        prod = (prod0, prod1)
        gsem = (gsem0, gsem1)
        osem = (osem0, osem1)

        def issue(c, parity):
            r = lax.shift_right_logical(c, 3)
            o = lax.mul(lax.bitwise_and(c, 7), 16)
            ublkv = ublk_v[r, pl.ds(o, 16)]
            iblkv = iblk_v[r, pl.ds(o, 16)]
            for i in range(_CHUNK):
                pltpu.async_copy(utab_hbm.at[pl.ds(ublkv[i] * 8, 8)],
                                 ub[parity].at[pl.ds(i * 8, 8)], gsem[parity])
                pltpu.async_copy(itab_hbm.at[pl.ds(iblkv[i] * 8, 8)],
                                 ib[parity].at[pl.ds(i * 8, 8)], gsem[parity])

        issue(jnp.int32(0), 0)

        def body(k, carry):
            for b in (0, 1):
                c = 2 * k + b
                bn = (b + 1) & 1
                cn = c + 1

                @pl.when(cn < n_chunks)
                def _():
                    issue(cn, bn)

                pltpu.make_async_copy(utab_hbm.at[pl.ds(0, _CHUNK * 8)],
                                      ub[b], gsem[b]).wait()
                pltpu.make_async_copy(itab_hbm.at[pl.ds(0, _CHUNK * 8)],
                                      ib[b], gsem[b]).wait()

                @pl.when(c >= 2)
                def _():
                    pltpu.make_async_copy(out_hbm.at[pl.ds(0, _CHUNK)],
                                          prod[b], osem[b]).wait()

                r = lax.shift_right_logical(c, 3)
                o = lax.mul(lax.bitwise_and(c, 7), 16)
                suv = usub_v[r, pl.ds(o, 16)]
                siv = isub_v[r, pl.ds(o, 16)]
                for i in range(_CHUNK):
                    su = suv[i]
                    si = siv[i]
                    for j in range(_EMBED // nl):
                        s = pl.ds(j * nl, nl)
                        prod[b][i, s] = ub[b][8 * i + su, s] * ib[b][8 * i + si, s]

                pltpu.async_copy(prod[b], out_hbm.at[pl.ds(base + c * _CHUNK,
                                                           _CHUNK)], osem[b])
            return carry

        lax.fori_loop(0, n_chunks // 2, body, 0)

        pltpu.make_async_copy(out_hbm.at[pl.ds(0, _CHUNK)], prod0, osem0).wait()
        pltpu.make_async_copy(out_hbm.at[pl.ds(0, _CHUNK)], prod1, osem1).wait()

    return gmf


_gmf = _build()


def kernel(user_indices, item_indices, user_table, item_table):
    uidx = user_indices.astype(jnp.int32)
    iidx = item_indices.astype(jnp.int32)
    ublk = (uidx >> 3).reshape(128, 128)
    usub = (uidx & 7).reshape(128, 128)
    iblk = (iidx >> 3).reshape(128, 128)
    isub = (iidx & 7).reshape(128, 128)
    return _gmf(ublk, usub, iblk, isub, user_table, item_table)


# speedup vs baseline: 1.7774x; 1.7774x over previous
import functools

import jax
import jax.numpy as jnp
from jax import lax
from jax.experimental import pallas as pl
from jax.experimental.pallas import tpu as pltpu
from jax.experimental.pallas import tpu_sc as plsc

_EMBED = 64
_BATCH = 16384
_ROWS = 1000000
_LANE = 128
_DEPTH = 4


def _build():
    info = plsc.get_sparse_core_info()
    nc, ns, nl = info.num_cores, info.num_subcores, info.num_lanes
    nw = nc * ns
    b_per_w = _BATCH // nw
    n_blocks = b_per_w // _LANE
    mesh = plsc.VectorSubcoreMesh(core_axis_name="c", subcore_axis_name="s")

    @functools.partial(
        pl.kernel,
        mesh=mesh,
        compiler_params=pltpu.CompilerParams(needs_layout_passes=False),
        out_type=jax.ShapeDtypeStruct((_EMBED, _BATCH), jnp.float32),
        scratch_types=[
            pltpu.VMEM((b_per_w + 32,), jnp.int32),
            pltpu.VMEM((b_per_w + 32,), jnp.int32),
            pltpu.VMEM((_DEPTH, _EMBED, _LANE), jnp.float32),
            pltpu.VMEM((_DEPTH, _EMBED, _LANE), jnp.float32),
            pltpu.VMEM((_EMBED, _LANE), jnp.float32),
            pltpu.VMEM((_EMBED, _LANE), jnp.float32),
            [pltpu.SemaphoreType.DMA] * _DEPTH,
            pltpu.SemaphoreType.DMA,
            pltpu.SemaphoreType.DMA,
        ],
    )
    def gmf(uidx_hbm, iidx_hbm, utab_hbm, itab_hbm, out_hbm,
            uidx_v, iidx_v, uw, iw, prod0, prod1, wsem, osem0, osem1):
        wid = lax.axis_index("s") * nc + lax.axis_index("c")
        base = wid * b_per_w
        pltpu.sync_copy(uidx_hbm.at[pl.ds(base, b_per_w)],
                        uidx_v.at[pl.ds(0, b_per_w)])
        pltpu.sync_copy(iidx_hbm.at[pl.ds(base, b_per_w)],
                        iidx_v.at[pl.ds(0, b_per_w)])

        prod = (prod0, prod1)
        osem = (osem0, osem1)
        iotas = tuple(lax.iota(jnp.int32, nl) + j * nl
                      for j in range(_EMBED // nl))

        def issue(u_idx, i_idx, slot):
            uo = pl.multiple_of(lax.shift_right_logical(u_idx, 7) * _LANE, 128)
            io = pl.multiple_of(lax.shift_right_logical(i_idx, 7) * _LANE, 128)
            pltpu.async_copy(utab_hbm.at[:, pl.ds(uo, _LANE)], uw.at[slot],
                             wsem[slot])
            pltpu.async_copy(itab_hbm.at[:, pl.ds(io, _LANE)], iw.at[slot],
                             wsem[slot])

        v0u = uidx_v[pl.ds(0, nl)]
        v0i = iidx_v[pl.ds(0, nl)]
        for s in range(_DEPTH):
            issue(v0u[s], v0i[s], s)

        def make_body(blk):
            p = blk & 1

            def body(k2, carry):
                k = blk * (_LANE // nl) + k2
                go = k * nl
                uv = uidx_v[pl.ds(go, nl)]
                un = uidx_v[pl.ds(go + nl, nl)]
                iv = iidx_v[pl.ds(go, nl)]
                inx = iidx_v[pl.ds(go + nl, nl)]
                for i in range(nl):
                    ci = go + i
                    s = i % _DEPTH
                    pltpu.make_async_copy(utab_hbm.at[:, pl.ds(0, _LANE)],
                                          uw.at[s], wsem[s]).wait()
                    pltpu.make_async_copy(itab_hbm.at[:, pl.ds(0, _LANE)],
                                          iw.at[s], wsem[s]).wait()
                    lu = jnp.full((nl,), lax.bitwise_and(uv[i], 127),
                                  jnp.int32)
                    li = jnp.full((nl,), lax.bitwise_and(iv[i], 127),
                                  jnp.int32)
                    lo = jnp.full((nl,), lax.bitwise_and(ci, 127), jnp.int32)
                    for j in range(_EMBED // nl):
                        gu = plsc.load_gather(uw.at[s], [iotas[j], lu])
                        gi = plsc.load_gather(iw.at[s], [iotas[j], li])
                        plsc.store_scatter(prod[p], [iotas[j], lo], gu * gi)
                    nu = un[(i + _DEPTH) % nl] if i + _DEPTH >= nl else uv[i + _DEPTH]
                    ni = inx[(i + _DEPTH) % nl] if i + _DEPTH >= nl else iv[i + _DEPTH]

                    @pl.when(ci + _DEPTH < b_per_w)
                    def _():
                        issue(nu, ni, s)
                return carry
            return body

        for blk in range(n_blocks):
            p = blk & 1
            if blk >= 2:
                pltpu.make_async_copy(out_hbm.at[:, pl.ds(0, _LANE)], prod[p],
                                      osem[p]).wait()
            lax.fori_loop(0, _LANE // nl, make_body(blk), 0)
            pltpu.async_copy(prod[p],
                             out_hbm.at[:, pl.ds(base + blk * _LANE, _LANE)],
                             osem[p])

        pltpu.make_async_copy(out_hbm.at[:, pl.ds(0, _LANE)], prod0,
                              osem0).wait()
        pltpu.make_async_copy(out_hbm.at[:, pl.ds(0, _LANE)], prod1,
                              osem1).wait()

    return gmf


_gmf = _build()


def kernel(user_indices, item_indices, user_table, item_table):
    uidx = user_indices.astype(jnp.int32)
    iidx = item_indices.astype(jnp.int32)
    outT = _gmf(uidx, iidx, user_table.T, item_table.T)
    return outT.T
